# baseline (speedup 1.0000x reference)
import functools

import jax
import jax.numpy as jnp
from jax import lax
from jax.experimental import pallas as pl
from jax.experimental.pallas import tpu as pltpu
from jax.experimental.pallas import tpu_sc as plsc

VOCAB = 1000000
MAXLEN = 200
EMBED = 64
BATCH = 4096

NUM_CORES = 2
NUM_SUBCORES = 16
LANES = 16
NUM_WORKERS = NUM_CORES * NUM_SUBCORES

PAD = 128
SEQS_PER_W = BATCH // NUM_WORKERS
NBUF = 2
NBLK = VOCAB // PAD


def _sc_prep(tt_t, tail):
    mesh = plsc.VectorSubcoreMesh(core_axis_name="c", subcore_axis_name="s")

    @functools.partial(
        pl.kernel,
        out_type=jax.ShapeDtypeStruct((VOCAB, PAD), jnp.float32),
        mesh=mesh,
        compiler_params=pltpu.CompilerParams(needs_layout_passes=False),
        scratch_types=[
            pltpu.VMEM((NBUF, EMBED, PAD), jnp.float32),
            pltpu.VMEM((NBUF, PAD, PAD), jnp.float32),
            pltpu.VMEM((EMBED, PAD), jnp.float32),
            pltpu.SemaphoreType.DMA,
            pltpu.SemaphoreType.DMA,
            pltpu.SemaphoreType.DMA,
            pltpu.SemaphoreType.DMA,
        ],
    )
    def tk(in_hbm, tail_hbm, out_hbm, i_v, o_v, tail_v, is0, is1, os0, os1):
        wid = lax.axis_index("s") * NUM_CORES + lax.axis_index("c")
        isems = (is0, is1)
        osems = (os0, os1)

        def start_of(i):
            return (wid + NUM_WORKERS * i) * PAD

        def valid(i):
            return wid + NUM_WORKERS * i < NBLK

        def in_start(i, slot):
            return pltpu.async_copy(
                in_hbm.at[:, pl.ds(start_of(i), PAD)], i_v.at[slot],
                isems[slot])

        def in_wait(i, slot):
            pltpu.make_async_copy(
                in_hbm.at[:, pl.ds(start_of(i), PAD)], i_v.at[slot],
                isems[slot]).wait()

        def out_start(i, slot):
            return pltpu.async_copy(
                o_v.at[slot], out_hbm.at[pl.ds(start_of(i), PAD)],
                osems[slot])

        def out_wait(i, slot):
            pltpu.make_async_copy(
                o_v.at[slot], out_hbm.at[pl.ds(start_of(i), PAD)],
                osems[slot]).wait()

        @pl.when(valid(0))
        def _():
            in_start(0, 0)

        nit = NBLK // NUM_WORKERS + 1

        @pl.loop(0, nit, step=NBUF)
        def _(i0):
            for b in range(NBUF):
                i = i0 + b
                other = 1 - b

                @pl.when(valid(i))
                def _():
                    @pl.when(i >= 1)
                    def _():
                        out_wait(i - 1, other)

                    @pl.when(valid(i + 1))
                    def _():
                        in_start(i + 1, other)

                    in_wait(i, b)

                    @pl.loop(0, PAD, unroll=8)
                    def _(t):
                        tv = jnp.full((LANES,), t, jnp.int32)
                        for g in range(EMBED // LANES):
                            ev = lax.iota(jnp.int32, LANES) + g * LANES
                            vals = plsc.load_gather(i_v.at[b], [ev, tv])
                            o_v[b, t, pl.ds(g * LANES, LANES)] = vals

                    out_start(i, b)

        @pl.when(wid == NUM_WORKERS - 1)
        def _():
            pltpu.sync_copy(tail_hbm, tail_v)
            pltpu.sync_copy(tail_v, out_hbm.at[pl.ds(NBLK * PAD, VOCAB - NBLK * PAD)])

        lastv = (NBLK - 1 - wid) // NUM_WORKERS
        for b in range(NBUF):
            @pl.when(lastv % NBUF == b)
            def _():
                out_wait(lastv, b)

    return tk(tt_t, tail)


def _sc_embed(x, t128, pos128):
    mesh = plsc.VectorSubcoreMesh(core_axis_name="c", subcore_axis_name="s")

    @functools.partial(
        pl.kernel,
        out_type=jax.ShapeDtypeStruct((BATCH, MAXLEN, PAD), jnp.float32),
        mesh=mesh,
        compiler_params=pltpu.CompilerParams(needs_layout_passes=False),
        scratch_types=[
            pltpu.VMEM((SEQS_PER_W * MAXLEN,), jnp.int32),
            pltpu.VMEM((NBUF, MAXLEN, PAD), jnp.float32),
            pltpu.VMEM((MAXLEN, PAD), jnp.float32),
            pltpu.SemaphoreType.DMA,
            pltpu.SemaphoreType.DMA,
            pltpu.SemaphoreType.DMA,
            pltpu.SemaphoreType.DMA,
        ],
    )
    def k(x_hbm, tok_hbm, pos_hbm, out_hbm, idx_all, g_v, pos_v,
          gsem0, gsem1, osem0, osem1):
        wid = lax.axis_index("s") * NUM_CORES + lax.axis_index("c")
        b0 = wid * SEQS_PER_W
        gsems = (gsem0, gsem1)
        osems = (osem0, osem1)

        pltpu.sync_copy(pos_hbm, pos_v)
        pltpu.sync_copy(x_hbm.at[pl.ds(b0 * MAXLEN, SEQS_PER_W * MAXLEN)], idx_all)

        def gather_start(j, slot):
            return pltpu.async_copy(
                tok_hbm.at[idx_all.at[pl.ds(j * MAXLEN, MAXLEN)]],
                g_v.at[slot], gsems[slot])

        def out_start(j, slot):
            return pltpu.async_copy(g_v.at[slot], out_hbm.at[b0 + j],
                                    osems[slot])

        def add_pos(slot):
            @pl.loop(0, MAXLEN, unroll=8)
            def _(r):
                for c in range(EMBED // LANES):
                    plsc.addupdate(
                        g_v.at[slot, r, pl.ds(c * LANES, LANES)],
                        pos_v[r, pl.ds(c * LANES, LANES)],
                    )

        GROUP = 16

        @pl.loop(0, SEQS_PER_W // GROUP)
        def _(gi):
            jb = gi * GROUP
            g = [None] * GROUP
            w = [None] * GROUP
            g[0] = gather_start(jb, 0)
            for t in range(GROUP):
                if t + 1 < GROUP:
                    if t >= 1:
                        w[t - 1].wait()
                    g[t + 1] = gather_start(jb + t + 1, (t + 1) % NBUF)
                g[t].wait()
                add_pos(t % NBUF)
                w[t] = out_start(jb + t, t % NBUF)
            w[GROUP - 2].wait()
            w[GROUP - 1].wait()

    return k(x, t128, pos128)


def kernel(x, token_table, pos_table):
    x32 = x.reshape(-1).astype(jnp.int32)
    tail = jnp.pad(token_table[NBLK * PAD:, :], ((0, 0), (0, PAD - EMBED)))
    t128 = _sc_prep(jnp.transpose(token_table), tail)
    pos128 = jnp.pad(pos_table, ((0, 0), (0, PAD - EMBED)))
    out = _sc_embed(x32, t128, pos128)
    return out[:, :, :EMBED]


# speedup vs baseline: 2.0797x; 2.0797x over previous
import functools

import jax
import jax.numpy as jnp
from jax import lax
from jax.experimental import pallas as pl
from jax.experimental.pallas import tpu as pltpu
from jax.experimental.pallas import tpu_sc as plsc

VOCAB = 1000000
MAXLEN = 200
EMBED = 64
BATCH = 4096

NUM_CORES = 2
NUM_SUBCORES = 16
LANES = 16
NUM_WORKERS = NUM_CORES * NUM_SUBCORES

PAD = 128
SEQS_PER_W = BATCH // NUM_WORKERS
NBUF = 2


def _sc_embed(x, t128, pos128):
    mesh = plsc.VectorSubcoreMesh(core_axis_name="c", subcore_axis_name="s")

    @functools.partial(
        pl.kernel,
        out_type=jax.ShapeDtypeStruct((BATCH, MAXLEN, PAD), jnp.float32),
        mesh=mesh,
        compiler_params=pltpu.CompilerParams(needs_layout_passes=False),
        scratch_types=[
            pltpu.VMEM((SEQS_PER_W * MAXLEN,), jnp.int32),
            pltpu.VMEM((NBUF, MAXLEN, PAD), jnp.float32),
            pltpu.VMEM((MAXLEN, PAD), jnp.float32),
            pltpu.SemaphoreType.DMA,
            pltpu.SemaphoreType.DMA,
            pltpu.SemaphoreType.DMA,
            pltpu.SemaphoreType.DMA,
        ],
    )
    def k(x_hbm, tok_hbm, pos_hbm, out_hbm, idx_all, g_v, pos_v,
          gsem0, gsem1, osem0, osem1):
        wid = lax.axis_index("s") * NUM_CORES + lax.axis_index("c")
        b0 = wid * SEQS_PER_W
        gsems = (gsem0, gsem1)
        osems = (osem0, osem1)

        pltpu.sync_copy(pos_hbm, pos_v)
        pltpu.sync_copy(x_hbm.at[pl.ds(b0 * MAXLEN, SEQS_PER_W * MAXLEN)], idx_all)

        def gather_start(j, slot):
            return pltpu.async_copy(
                tok_hbm.at[idx_all.at[pl.ds(j * MAXLEN, MAXLEN)]],
                g_v.at[slot], gsems[slot])

        def out_start(j, slot):
            return pltpu.async_copy(g_v.at[slot], out_hbm.at[b0 + j],
                                    osems[slot])

        def add_pos(slot):
            @pl.loop(0, MAXLEN, unroll=8)
            def _(r):
                for c in range(EMBED // LANES):
                    plsc.addupdate(
                        g_v.at[slot, r, pl.ds(c * LANES, LANES)],
                        pos_v[r, pl.ds(c * LANES, LANES)],
                    )

        GROUP = 16

        @pl.loop(0, SEQS_PER_W // GROUP)
        def _(gi):
            jb = gi * GROUP
            g = [None] * GROUP
            w = [None] * GROUP
            g[0] = gather_start(jb, 0)
            for t in range(GROUP):
                if t + 1 < GROUP:
                    if t >= 1:
                        w[t - 1].wait()
                    g[t + 1] = gather_start(jb + t + 1, (t + 1) % NBUF)
                g[t].wait()
                add_pos(t % NBUF)
                w[t] = out_start(jb + t, t % NBUF)
            w[GROUP - 2].wait()
            w[GROUP - 1].wait()

    return k(x, t128, pos128)


def kernel(x, token_table, pos_table):
    x32 = x.reshape(-1).astype(jnp.int32)
    t128 = jnp.pad(token_table, ((0, 0), (0, PAD - EMBED)))
    pos128 = jnp.pad(pos_table, ((0, 0), (0, PAD - EMBED)))
    out = _sc_embed(x32, t128, pos128)
    return out[:, :, :EMBED]
